# baseline (speedup 1.0000x reference)
import jax
import jax.numpy as jnp
from jax import lax
from jax.experimental import pallas as pl
from jax.experimental.pallas import tpu as pltpu


def _gate_body(x_ref, wg_ref, g1_ref, g2_ref, i1_ref, i2_ref, l1_ref, l2_ref,
               loss_ref, cnt1_ref, cnt2_ref, me_ref, i2s_ref, l2p_ref,
               tri_ref):
    step = pl.program_id(0)
    nb = pl.num_programs(0) - 1
    bt = x_ref.shape[0]
    ne = wg_ref.shape[0]

    @pl.when(step == 0)
    def _init():
        cnt1_ref[...] = jnp.zeros_like(cnt1_ref)
        cnt2_ref[...] = jnp.zeros_like(cnt2_ref)
        me_ref[...] = jnp.zeros_like(me_ref)
        r = lax.broadcasted_iota(jnp.int32, (bt, bt), 0)
        c = lax.broadcasted_iota(jnp.int32, (bt, bt), 1)
        tri_ref[...] = (c <= r).astype(jnp.bfloat16)

    @pl.when(step < nb)
    def _main():
        logits = lax.dot_general(
            x_ref[...].astype(jnp.bfloat16), wg_ref[...].astype(jnp.bfloat16),
            (((1,), (1,)), ((), ())), preferred_element_type=jnp.float32)

        col = lax.broadcasted_iota(jnp.int32, (bt, ne), 1)
        m1 = jnp.max(logits, axis=1, keepdims=True)
        i1 = jnp.min(jnp.where(logits == m1, col, ne), axis=1)
        onehot1 = col == i1[:, None]
        masked = jnp.where(onehot1, -jnp.inf, logits)
        m2 = jnp.max(masked, axis=1, keepdims=True)
        i2 = jnp.min(jnp.where(masked == m2, col, ne), axis=1)
        onehot2 = col == i2[:, None]

        p = jnp.exp(logits - m1)
        z = jnp.sum(p, axis=1, keepdims=True)
        g1 = 1.0 / z
        g2 = jnp.exp(m2 - m1) / z
        den = jnp.maximum(g1 + g2, jnp.finfo(jnp.float32).eps)
        g1_ref[...] = (g1 / den)[:, 0]
        g2_ref[...] = (g2 / den)[:, 0]
        i1_ref[...] = i1
        i2_ref[...] = i2

        tri = tri_ref[...]
        cum1 = lax.dot_general(tri, onehot1.astype(jnp.bfloat16),
                               (((1,), (0,)), ((), ())),
                               preferred_element_type=jnp.float32)
        cum2 = lax.dot_general(tri, onehot2.astype(jnp.bfloat16),
                               (((1,), (0,)), ((), ())),
                               preferred_element_type=jnp.float32)
        c1 = cnt1_ref[...]
        c2 = cnt2_ref[...]
        l1_ref[...] = jnp.sum(jnp.where(onehot1, cum1 - 1.0 + c1, 0.0),
                              axis=1).astype(jnp.int32)
        loc2p = jnp.sum(jnp.where(onehot2, cum2 - 1.0 + c2, 0.0),
                        axis=1).astype(jnp.int32)
        i2s_ref[pl.ds(step, 1), :] = i2[None, :]
        l2p_ref[pl.ds(step, 1), :] = loc2p[None, :]
        cnt1_ref[...] = c1 + cum1[bt - 1:bt, :]
        cnt2_ref[...] = c2 + cum2[bt - 1:bt, :]
        me_ref[...] = me_ref[...] + jnp.sum(p / z, axis=0, keepdims=True)

    @pl.when(step == nb)
    def _fin():
        ntok = nb * bt
        ce = cnt1_ref[...]
        loss_ref[...] = (jnp.sum(me_ref[...] * ce) * (ne / (ntok * ntok))
                         ).reshape(1, 1)
        col = lax.broadcasted_iota(jnp.int32, (bt, ne), 1)
        for cblk in range(nb):
            i2c = i2s_ref[cblk, :]
            oh = (col == i2c[:, None]).astype(jnp.float32)
            add = lax.dot_general(oh, ce, (((1,), (1,)), ((), ())),
                                  preferred_element_type=jnp.float32)
            l2_ref[pl.ds(cblk * bt, bt)] = (
                l2p_ref[cblk, :] + add[:, 0].astype(jnp.int32))


def _gate_call(x, wg, bt):
    nt, d = x.shape
    ne = wg.shape[0]
    nb = nt // bt
    tok = pl.BlockSpec((bt,), lambda i: (jnp.minimum(i, nb - 1),))
    return pl.pallas_call(
        _gate_body,
        grid=(nb + 1,),
        in_specs=[pl.BlockSpec((bt, d), lambda i: (jnp.minimum(i, nb - 1), 0)),
                  pl.BlockSpec((ne, d), lambda i: (0, 0))],
        out_specs=[tok, tok, tok, tok, tok,
                   pl.BlockSpec((nt,), lambda i: (0,)),
                   pl.BlockSpec((1, 1), lambda i: (0, 0))],
        out_shape=[jax.ShapeDtypeStruct((nt,), jnp.float32),
                   jax.ShapeDtypeStruct((nt,), jnp.float32),
                   jax.ShapeDtypeStruct((nt,), jnp.int32),
                   jax.ShapeDtypeStruct((nt,), jnp.int32),
                   jax.ShapeDtypeStruct((nt,), jnp.int32),
                   jax.ShapeDtypeStruct((nt,), jnp.int32),
                   jax.ShapeDtypeStruct((1, 1), jnp.float32)],
        scratch_shapes=[pltpu.VMEM((1, ne), jnp.float32),
                        pltpu.VMEM((1, ne), jnp.float32),
                        pltpu.VMEM((1, ne), jnp.float32),
                        pltpu.VMEM((nb, bt), jnp.int32),
                        pltpu.VMEM((nb, bt), jnp.int32),
                        pltpu.VMEM((bt, bt), jnp.bfloat16)],
    )(x, wg)


def kernel(x, wg):
    g1, g2, i1, i2, loc1, loc2, loss = _gate_call(x, wg, bt=1024)
    return loss.reshape(()), g1, g2, i1, i2, loc1, loc2


# speedup vs baseline: 1.1817x; 1.1817x over previous
import jax
import jax.numpy as jnp
from jax import lax
from jax.experimental import pallas as pl
from jax.experimental.pallas import tpu as pltpu


def _gate_body(x_ref, wg_ref, g1_ref, g2_ref, i1_ref, i2_ref, l1_ref, l2_ref,
               loss_ref, cnt1_ref, cnt2_ref, me_ref, i2s_ref, l2p_ref,
               tri_ref):
    step = pl.program_id(0)
    nb = pl.num_programs(0) - 1
    bt = x_ref.shape[0]
    ne = wg_ref.shape[0]

    @pl.when(step == 0)
    def _init():
        cnt1_ref[...] = jnp.zeros_like(cnt1_ref)
        cnt2_ref[...] = jnp.zeros_like(cnt2_ref)
        me_ref[...] = jnp.zeros_like(me_ref)
        r = lax.broadcasted_iota(jnp.int32, (bt, bt), 0)
        c = lax.broadcasted_iota(jnp.int32, (bt, bt), 1)
        tri_ref[...] = (c <= r).astype(jnp.bfloat16)

    @pl.when(step < nb)
    def _main():
        logits = lax.dot_general(
            x_ref[...].astype(jnp.bfloat16), wg_ref[...].astype(jnp.bfloat16),
            (((1,), (1,)), ((), ())), preferred_element_type=jnp.float32)

        col = lax.broadcasted_iota(jnp.int32, (bt, ne), 1)
        m1 = jnp.max(logits, axis=1, keepdims=True)
        i1 = jnp.min(jnp.where(logits == m1, col, ne), axis=1)
        onehot1 = col == i1[:, None]
        masked = jnp.where(onehot1, -jnp.inf, logits)
        m2 = jnp.max(masked, axis=1, keepdims=True)
        i2 = jnp.min(jnp.where(masked == m2, col, ne), axis=1)
        onehot2 = col == i2[:, None]

        p = jnp.exp(logits - m1)
        z = jnp.sum(p, axis=1, keepdims=True)
        g1 = 1.0 / z
        g2 = jnp.exp(m2 - m1) / z
        den = jnp.maximum(g1 + g2, jnp.finfo(jnp.float32).eps)
        g1_ref[...] = (g1 / den)[:, 0]
        g2_ref[...] = (g2 / den)[:, 0]
        i1_ref[...] = i1
        i2_ref[...] = i2

        tri = tri_ref[...]
        cum1 = lax.dot_general(tri, onehot1.astype(jnp.bfloat16),
                               (((1,), (0,)), ((), ())),
                               preferred_element_type=jnp.float32)
        cum2 = lax.dot_general(tri, onehot2.astype(jnp.bfloat16),
                               (((1,), (0,)), ((), ())),
                               preferred_element_type=jnp.float32)
        c1 = cnt1_ref[...]
        c2 = cnt2_ref[...]
        l1_ref[...] = jnp.sum(jnp.where(onehot1, cum1 - 1.0 + c1, 0.0),
                              axis=1).astype(jnp.int32)
        loc2p = jnp.sum(jnp.where(onehot2, cum2 - 1.0 + c2, 0.0),
                        axis=1).astype(jnp.int32)
        i2s_ref[pl.ds(step, 1), :] = i2[None, :]
        l2p_ref[pl.ds(step, 1), :] = loc2p[None, :]
        cnt1_ref[...] = c1 + cum1[bt - 1:bt, :]
        cnt2_ref[...] = c2 + cum2[bt - 1:bt, :]
        me_ref[...] = me_ref[...] + jnp.sum(p / z, axis=0, keepdims=True)

    @pl.when(step == nb)
    def _fin():
        ntok = nb * bt
        ce = cnt1_ref[...]
        loss_ref[...] = (jnp.sum(me_ref[...] * ce) * (ne / (ntok * ntok))
                         ).reshape(1, 1)
        col = lax.broadcasted_iota(jnp.int32, (bt, ne), 1)
        for cblk in range(nb):
            i2c = i2s_ref[cblk, :]
            oh = (col == i2c[:, None]).astype(jnp.float32)
            add = lax.dot_general(oh, ce, (((1,), (1,)), ((), ())),
                                  preferred_element_type=jnp.float32)
            l2_ref[pl.ds(cblk * bt, bt)] = (
                l2p_ref[cblk, :] + add[:, 0].astype(jnp.int32))


def _gate_call(x, wg, bt):
    nt, d = x.shape
    ne = wg.shape[0]
    nb = nt // bt
    tok = pl.BlockSpec((bt,), lambda i: (jnp.minimum(i, nb - 1),))
    return pl.pallas_call(
        _gate_body,
        grid=(nb + 1,),
        in_specs=[pl.BlockSpec((bt, d), lambda i: (jnp.minimum(i, nb - 1), 0)),
                  pl.BlockSpec((ne, d), lambda i: (0, 0))],
        out_specs=[tok, tok, tok, tok, tok,
                   pl.BlockSpec((nt,), lambda i: (0,)),
                   pl.BlockSpec((1, 1), lambda i: (0, 0))],
        out_shape=[jax.ShapeDtypeStruct((nt,), jnp.float32),
                   jax.ShapeDtypeStruct((nt,), jnp.float32),
                   jax.ShapeDtypeStruct((nt,), jnp.int32),
                   jax.ShapeDtypeStruct((nt,), jnp.int32),
                   jax.ShapeDtypeStruct((nt,), jnp.int32),
                   jax.ShapeDtypeStruct((nt,), jnp.int32),
                   jax.ShapeDtypeStruct((1, 1), jnp.float32)],
        scratch_shapes=[pltpu.VMEM((1, ne), jnp.float32),
                        pltpu.VMEM((1, ne), jnp.float32),
                        pltpu.VMEM((1, ne), jnp.float32),
                        pltpu.VMEM((nb, bt), jnp.int32),
                        pltpu.VMEM((nb, bt), jnp.int32),
                        pltpu.VMEM((bt, bt), jnp.bfloat16)],
    )(x, wg)


def kernel(x, wg):
    g1, g2, i1, i2, loc1, loc2, loss = _gate_call(x, wg, bt=256)
    return loss.reshape(()), g1, g2, i1, i2, loc1, loc2
